# Initial kernel scaffold; baseline (speedup 1.0000x reference)
import jax
import jax.numpy as jnp
from jax.experimental import pallas as pl


def kernel(x, table):
    raise NotImplementedError("write your pallas kernel here")


# speedup vs baseline: 1.8967x; 1.8967x over previous
import functools

import jax
import jax.numpy as jnp
from jax import lax
from jax.experimental import pallas as pl
from jax.experimental.pallas import tpu as pltpu
from jax.experimental.pallas import tpu_sc as plsc

NUM_EMB = 1000000
B = 16384
L = 50
D = 32
NC = 2
NS = 16
NW = NC * NS
BPW = B // NW


def _make_sc_kernel():
    mesh = plsc.VectorSubcoreMesh(core_axis_name="c", subcore_axis_name="s")

    @functools.partial(
        pl.kernel,
        mesh=mesh,
        out_type=jax.ShapeDtypeStruct((B, D), jnp.float32),
        compiler_params=pltpu.CompilerParams(use_tc_tiling_on_sc=False),
        scratch_types=[
            pltpu.VMEM((BPW, L), jnp.int32),
            pltpu.VMEM((L, D), jnp.float32),
            pltpu.VMEM((BPW, D), jnp.float32),
            pltpu.SemaphoreType.DMA,
        ],
    )
    def k(idx_hbm, table_hbm, out_hbm, idx_v, buf_v, out_v, sem):
        wid = lax.axis_index("s") * NC + lax.axis_index("c")
        base = wid * BPW
        pltpu.sync_copy(idx_hbm.at[pl.ds(base, BPW)], idx_v)

        def row_body(s, carry):
            pltpu.async_copy(table_hbm.at[idx_v.at[s]], buf_v, sem).wait()
            a0 = buf_v[0, pl.ds(0, 16)]
            a1 = buf_v[0, pl.ds(16, 16)]
            for j in range(1, L):
                a0 = a0 + buf_v[j, pl.ds(0, 16)]
                a1 = a1 + buf_v[j, pl.ds(16, 16)]
            out_v[s, pl.ds(0, 16)] = a0
            out_v[s, pl.ds(16, 16)] = a1
            return carry

        lax.fori_loop(0, BPW, row_body, 0)
        pltpu.sync_copy(out_v, out_hbm.at[pl.ds(base, BPW)])

    return k


_sc_kernel = _make_sc_kernel()


def kernel(x, table):
    idx = x.astype(jnp.int32)
    return _sc_kernel(idx, table)


# speedup vs baseline: 2.8905x; 1.5240x over previous
import functools

import jax
import jax.numpy as jnp
from jax import lax
from jax.experimental import pallas as pl
from jax.experimental.pallas import tpu as pltpu
from jax.experimental.pallas import tpu_sc as plsc

NUM_EMB = 1000000
B = 16384
L = 50
D = 32
NC = 2
NS = 16
NW = NC * NS
BPW = B // NW
NBUF = 8


def _make_sc_kernel():
    mesh = plsc.VectorSubcoreMesh(core_axis_name="c", subcore_axis_name="s")

    @functools.partial(
        pl.kernel,
        mesh=mesh,
        out_type=jax.ShapeDtypeStruct((B, D), jnp.float32),
        compiler_params=pltpu.CompilerParams(use_tc_tiling_on_sc=False),
        scratch_types=[
            pltpu.VMEM((BPW, L), jnp.int32),
            pltpu.VMEM((NBUF, L, D), jnp.float32),
            pltpu.VMEM((BPW, D), jnp.float32),
            pltpu.SemaphoreType.DMA((NBUF,)),
        ],
    )
    def k(idx_hbm, table_hbm, out_hbm, idx_v, buf_v, out_v, sems):
        wid = lax.axis_index("s") * NC + lax.axis_index("c")
        base = wid * BPW
        pltpu.sync_copy(idx_hbm.at[pl.ds(base, BPW)], idx_v)

        for b in range(NBUF):
            pltpu.async_copy(table_hbm.at[idx_v.at[b]], buf_v.at[b], sems.at[b])

        def group_body(g, carry):
            for b in range(NBUF):
                s = g * NBUF + b
                pltpu.make_async_copy(
                    table_hbm.at[idx_v.at[0]], buf_v.at[b], sems.at[b]
                ).wait()
                a0 = buf_v[b, 0, pl.ds(0, 16)]
                a1 = buf_v[b, 0, pl.ds(16, 16)]
                for j in range(1, L):
                    a0 = a0 + buf_v[b, j, pl.ds(0, 16)]
                    a1 = a1 + buf_v[b, j, pl.ds(16, 16)]
                out_v[s, pl.ds(0, 16)] = a0
                out_v[s, pl.ds(16, 16)] = a1
                nxt = s + NBUF

                @pl.when(nxt < BPW)
                def _():
                    pltpu.async_copy(
                        table_hbm.at[idx_v.at[nxt]], buf_v.at[b], sems.at[b]
                    )

            return carry

        lax.fori_loop(0, BPW // NBUF, group_body, 0)
        pltpu.sync_copy(out_v, out_hbm.at[pl.ds(base, BPW)])

    return k


_sc_kernel = _make_sc_kernel()


def kernel(x, table):
    idx = x.astype(jnp.int32)
    return _sc_kernel(idx, table)
